# Initial kernel scaffold; baseline (speedup 1.0000x reference)
import jax
import jax.numpy as jnp
from jax.experimental import pallas as pl


def kernel(atom_emb, atom_mean_tab, bond_emb, msg_params, out_W, out_b, final_W, final_b, atom_types, bond_types, node_graph_indices, connectivity):
    raise NotImplementedError("write your pallas kernel here")


# speedup vs baseline: 2.1527x; 2.1527x over previous
import jax
import jax.numpy as jnp
from jax import lax
from jax.experimental import pallas as pl
from jax.experimental.pallas import tpu as pltpu
from jax.experimental.pallas import tpu_sc as plsc

N = 10000
E = 320000
D = 128
A = 100
BC = 20
G = 256
H = 128
EPS = 1e-3

NC = 2
NS = 16
NW = NC * NS

NP = 10240
ZR = NP // NS
GC = 400
GCS = 200
BE = 2000
BN = 1000


def _gather_body(table, src_idx, dst_idx, src_out, dst_out, idx_v, rows_v, sem):
    wid = lax.axis_index("s") * NC + lax.axis_index("c")
    epw = E // NW
    base = wid * epw

    def step(j, carry):
        off = base + j * GC
        pltpu.sync_copy(src_idx.at[pl.ds(off, GC)], idx_v)
        pltpu.async_copy(table.at[idx_v], rows_v, sem).wait()
        pltpu.sync_copy(rows_v, src_out.at[pl.ds(off, GC)])
        pltpu.sync_copy(dst_idx.at[pl.ds(off, GC)], idx_v)
        pltpu.async_copy(table.at[idx_v], rows_v, sem).wait()
        pltpu.sync_copy(rows_v, dst_out.at[pl.ds(off, GC)])
        return carry

    lax.fori_loop(0, epw // GC, step, 0)


def _sc_gather(table, src_idx, dst_idx):
    mesh = plsc.VectorSubcoreMesh(core_axis_name="c", subcore_axis_name="s")
    return pl.kernel(
        _gather_body,
        out_type=(
            jax.ShapeDtypeStruct((E, D), jnp.float32),
            jax.ShapeDtypeStruct((E, D), jnp.float32),
        ),
        mesh=mesh,
        scratch_types=[
            pltpu.VMEM((GC,), jnp.int32),
            pltpu.VMEM((GC, D), jnp.float32),
            pltpu.SemaphoreType.DMA,
        ],
    )(table, src_idx, dst_idx)


def _scatter_body(msg, dst_idx, zblk, p0, p1, acc, idx_v, rows_v):
    c = lax.axis_index("c")
    s = lax.axis_index("s")
    pltpu.sync_copy(zblk, acc.at[pl.ds(s * ZR, ZR)])
    plsc.subcore_barrier()

    half = E // NC
    epw = half // NS
    base = c * half + s * epw

    def step(j, carry):
        off = base + j * GCS
        pltpu.sync_copy(dst_idx.at[pl.ds(off, GCS)], idx_v)
        pltpu.sync_copy(msg.at[pl.ds(off, GCS)], rows_v)
        pltpu.sync_copy(rows_v, acc.at[idx_v], add=True)
        return carry

    lax.fori_loop(0, epw // GCS, step, 0)
    plsc.subcore_barrier()

    @pl.when(c == 0)
    def _():
        pltpu.sync_copy(acc.at[pl.ds(s * ZR, ZR)], p0.at[pl.ds(s * ZR, ZR)])

    @pl.when(c == 1)
    def _():
        pltpu.sync_copy(acc.at[pl.ds(s * ZR, ZR)], p1.at[pl.ds(s * ZR, ZR)])


def _sc_scatter(msg, dst_idx, zblk):
    mesh = plsc.VectorSubcoreMesh(core_axis_name="c", subcore_axis_name="s")
    return pl.kernel(
        _scatter_body,
        out_type=(
            jax.ShapeDtypeStruct((NP, D), jnp.float32),
            jax.ShapeDtypeStruct((NP, D), jnp.float32),
        ),
        mesh=mesh,
        scratch_types=[
            pltpu.VMEM_SHARED((NP, D), jnp.float32),
            pltpu.VMEM((GCS,), jnp.int32),
            pltpu.VMEM((GCS, D), jnp.float32),
        ],
    )(msg, dst_idx, zblk)


def _init_kernel_body(types_ref, emb_ref, out_ref):
    t = types_ref[...]
    oh = (t == lax.broadcasted_iota(jnp.int32, (BN, A), 1)).astype(jnp.float32)
    out_ref[...] = jnp.dot(oh, emb_ref[...], preferred_element_type=jnp.float32)


def _tc_init(atom_types_2d, atom_emb):
    return pl.pallas_call(
        _init_kernel_body,
        grid=(N // BN,),
        in_specs=[
            pl.BlockSpec((BN, 1), lambda i: (i, 0)),
            pl.BlockSpec((A, D), lambda i: (0, 0)),
        ],
        out_specs=pl.BlockSpec((BN, D), lambda i: (i, 0)),
        out_shape=jax.ShapeDtypeStruct((N, D), jnp.float32),
    )(atom_types_2d, atom_emb)


def _edge_compute(src, tgt, bond, W1s, W1t, W1b, W2, Wa, hb, ab, b2,
                  msg_ref, bond_ref):
    hid = jnp.tanh(
        jnp.dot(src, W1s, preferred_element_type=jnp.float32)
        + jnp.dot(tgt, W1t, preferred_element_type=jnp.float32)
        + jnp.dot(bond, W1b, preferred_element_type=jnp.float32)
        + hb)
    nb = jnp.dot(hid, W2, preferred_element_type=jnp.float32) + b2
    u = jnp.tanh(jnp.dot(src, Wa, preferred_element_type=jnp.float32) + ab)
    msg_ref[...] = u * nb
    bond_ref[...] = bond + nb


def _edge_body_first(src_ref, tgt_ref, btyp_ref, bemb_ref, W1s_ref, W1t_ref,
                     W1b_ref, W2_ref, Wa_ref, hb_ref, ab_ref, b2_ref,
                     msg_ref, bond_ref):
    bt = btyp_ref[...]
    oh = (bt == lax.broadcasted_iota(jnp.int32, (BE, BC), 1)).astype(jnp.float32)
    bond = jnp.dot(oh, bemb_ref[...], preferred_element_type=jnp.float32)
    _edge_compute(src_ref[...], tgt_ref[...], bond, W1s_ref[...],
                  W1t_ref[...], W1b_ref[...], W2_ref[...], Wa_ref[...],
                  hb_ref[...], ab_ref[...], b2_ref[...], msg_ref, bond_ref)


def _edge_body_next(src_ref, tgt_ref, bin_ref, W1s_ref, W1t_ref,
                    W1b_ref, W2_ref, Wa_ref, hb_ref, ab_ref, b2_ref,
                    msg_ref, bond_ref):
    _edge_compute(src_ref[...], tgt_ref[...], bin_ref[...],
                  W1s_ref[...], W1t_ref[...], W1b_ref[...], W2_ref[...],
                  Wa_ref[...], hb_ref[...], ab_ref[...], b2_ref[...],
                  msg_ref, bond_ref)


def _full_spec(shape):
    return pl.BlockSpec(shape, lambda i: tuple(0 for _ in shape))


def _tc_edge(first, src_rows, dst_rows, bond_in, weights):
    body = _edge_body_first if first else _edge_body_next
    eb = pl.BlockSpec((BE, D), lambda i: (i, 0))
    if first:
        data_specs = [eb, eb, pl.BlockSpec((BE, 1), lambda i: (i, 0)),
                      _full_spec((BC, D))]
    else:
        data_specs = [eb, eb, eb]
    w_specs = [_full_spec((D, 2 * D)), _full_spec((D, 2 * D)),
               _full_spec((D, 2 * D)), _full_spec((2 * D, D)),
               _full_spec((D, D)), _full_spec((1, 2 * D)),
               _full_spec((1, D)), _full_spec((1, D))]
    return pl.pallas_call(
        body,
        grid=(E // BE,),
        in_specs=data_specs + w_specs,
        out_specs=(eb, eb),
        out_shape=(
            jax.ShapeDtypeStruct((E, D), jnp.float32),
            jax.ShapeDtypeStruct((E, D), jnp.float32),
        ),
    )(src_rows, dst_rows, *bond_in, *weights)


def _combine_body(a_ref, p0_ref, p1_ref, out_ref):
    out_ref[...] = a_ref[...] + p0_ref[...] + p1_ref[...]


def _tc_combine(atom, p0, p1):
    nb = pl.BlockSpec((BE, D), lambda i: (i, 0))
    return pl.pallas_call(
        _combine_body,
        grid=(N // BE,),
        in_specs=[nb, nb, nb],
        out_specs=nb,
        out_shape=jax.ShapeDtypeStruct((N, D), jnp.float32),
    )(atom, p0, p1)


def _final_body(a_ref, p0_ref, p1_ref, typ_ref, ng_ref, outW_ref, outb_ref,
                finW_ref, finb_ref, mtab_ref, out_ref):
    x = a_ref[...] + p0_ref[...] + p1_ref[...]
    h = jnp.maximum(
        jnp.dot(x, outW_ref[...], preferred_element_type=jnp.float32)
        + outb_ref[...], 0.0)
    e = jnp.dot(h, finW_ref[...], preferred_element_type=jnp.float32) + finb_ref[...]
    t = typ_ref[...]
    ohm = (t == lax.broadcasted_iota(jnp.int32, (BN, A), 1)).astype(jnp.float32)
    e = e + jnp.dot(ohm, mtab_ref[...], preferred_element_type=jnp.float32)
    g_row = ng_ref[...].reshape(1, BN)
    ohg = (jnp.broadcast_to(g_row, (G, BN))
           == lax.broadcasted_iota(jnp.int32, (G, BN), 0)).astype(jnp.float32)
    partial = jnp.dot(ohg, e, preferred_element_type=jnp.float32)

    @pl.when(pl.program_id(0) == 0)
    def _():
        out_ref[...] = jnp.zeros_like(out_ref)

    out_ref[...] += partial


def _tc_final(atom, p0, p1, atom_types_2d, ng_row, out_W, out_b2, final_W,
              final_b2, mtab):
    nb = pl.BlockSpec((BN, D), lambda i: (i, 0))
    return pl.pallas_call(
        _final_body,
        grid=(N // BN,),
        in_specs=[nb, nb, nb,
                  pl.BlockSpec((BN, 1), lambda i: (i, 0)),
                  pl.BlockSpec((1, 1, BN), lambda i: (i, 0, 0)),
                  _full_spec((D, H)), _full_spec((1, H)), _full_spec((H, 1)),
                  _full_spec((1, 1)), _full_spec((A, 1))],
        out_specs=pl.BlockSpec((G, 1), lambda i: (0, 0)),
        out_shape=jax.ShapeDtypeStruct((G, 1), jnp.float32),
    )(atom, p0, p1, atom_types_2d, ng_row, out_W, out_b2, final_W, final_b2,
      mtab)


def kernel(atom_emb, atom_mean_tab, bond_emb, msg_params, out_W, out_b,
           final_W, final_b, atom_types, bond_types, node_graph_indices,
           connectivity):
    inv = 1.0 / jnp.sqrt(1.0 + EPS)
    dst_idx = connectivity[:, 0].astype(jnp.int32)
    src_idx = connectivity[:, 1].astype(jnp.int32)
    atom_types_2d = atom_types.astype(jnp.int32).reshape(N, 1)
    bond_types_2d = bond_types.astype(jnp.int32).reshape(E, 1)
    ng_row = node_graph_indices.astype(jnp.int32).reshape(N // BN, 1, BN)
    zblk = jnp.zeros((ZR, D), jnp.float32)

    step_weights = []
    for p in msg_params:
        sa = p['atom_bn_gamma'] * inv
        ba = p['atom_bn_beta']
        sb = p['bond_bn_gamma'] * inv
        bb = p['bond_bn_beta']
        W1 = p['W1']
        W1s = sa[:, None] * W1[:D]
        W1t = sa[:, None] * W1[D:2 * D]
        W1b = sb[:, None] * W1[2 * D:]
        hb = (ba @ (W1[:D] + W1[D:2 * D]) + bb @ W1[2 * D:]).reshape(1, 2 * D)
        Wa = sa[:, None] * p['Wa']
        ab = (ba @ p['Wa']).reshape(1, D)
        b2 = p['b2'].reshape(1, D)
        step_weights.append((W1s, W1t, W1b, p['W2'], Wa, hb, ab, b2))

    atom = _tc_init(atom_types_2d, atom_emb)
    bond = None
    p0 = p1 = None
    for t in range(len(msg_params)):
        src_rows, dst_rows = _sc_gather(atom, src_idx, dst_idx)
        if t == 0:
            msg, bond = _tc_edge(True, src_rows, dst_rows,
                                 (bond_types_2d, bond_emb), step_weights[t])
        else:
            msg, bond = _tc_edge(False, src_rows, dst_rows, (bond,),
                                 step_weights[t])
        p0, p1 = _sc_scatter(msg, dst_idx, zblk)
        if t < len(msg_params) - 1:
            atom = _tc_combine(atom, p0, p1)

    return _tc_final(atom, p0, p1, atom_types_2d, ng_row, out_W,
                     out_b.reshape(1, H), final_W, final_b.reshape(1, 1),
                     atom_mean_tab)


# speedup vs baseline: 2.3565x; 1.0947x over previous
import jax
import jax.numpy as jnp
from jax import lax
from jax.experimental import pallas as pl
from jax.experimental.pallas import tpu as pltpu
from jax.experimental.pallas import tpu_sc as plsc

N = 10000
E = 320000
D = 128
A = 100
BC = 20
G = 256
H = 128
EPS = 1e-3

NC = 2
NS = 16
NW = NC * NS

NP = 10240
ZR = NP // NS
GC = 400
GCS = 200
BE = 2000
BN = 1000


def _gather_body(table, src_idx, dst_idx, src_out, dst_out, idxs_v, idxt_v,
                 r0, r1, s0, s1, w0, w1):
    wid = lax.axis_index("s") * NC + lax.axis_index("c")
    epw = E // NW
    base = wid * epw
    pltpu.sync_copy(src_idx.at[pl.ds(base, epw)], idxs_v)
    pltpu.sync_copy(dst_idx.at[pl.ds(base, epw)], idxt_v)

    def step(j, carry):
        off = base + j * GC
        cs = pltpu.async_copy(table.at[idxs_v.at[pl.ds(j * GC, GC)]], r0, s0)
        ct = pltpu.async_copy(table.at[idxt_v.at[pl.ds(j * GC, GC)]], r1, s1)
        cs.wait()
        ws = pltpu.async_copy(r0, src_out.at[pl.ds(off, GC)], w0)
        ct.wait()
        wt = pltpu.async_copy(r1, dst_out.at[pl.ds(off, GC)], w1)
        ws.wait()
        wt.wait()
        return carry

    lax.fori_loop(0, epw // GC, step, 0)


def _sc_gather(table, src_idx, dst_idx):
    mesh = plsc.VectorSubcoreMesh(core_axis_name="c", subcore_axis_name="s")
    epw = E // NW
    return pl.kernel(
        _gather_body,
        out_type=(
            jax.ShapeDtypeStruct((E, D), jnp.float32),
            jax.ShapeDtypeStruct((E, D), jnp.float32),
        ),
        mesh=mesh,
        scratch_types=[
            pltpu.VMEM((epw,), jnp.int32),
            pltpu.VMEM((epw,), jnp.int32),
            pltpu.VMEM((GC, D), jnp.float32),
            pltpu.VMEM((GC, D), jnp.float32),
            pltpu.SemaphoreType.DMA,
            pltpu.SemaphoreType.DMA,
            pltpu.SemaphoreType.DMA,
            pltpu.SemaphoreType.DMA,
        ],
    )(table, src_idx, dst_idx)


def _scatter_body(msg, dst_idx, zblk, p0, p1, acc, idx_v, rows_v):
    c = lax.axis_index("c")
    s = lax.axis_index("s")
    pltpu.sync_copy(zblk, acc.at[pl.ds(s * ZR, ZR)])
    plsc.subcore_barrier()

    half = E // NC
    epw = half // NS
    base = c * half + s * epw

    def step(j, carry):
        off = base + j * GCS
        pltpu.sync_copy(dst_idx.at[pl.ds(off, GCS)], idx_v)
        pltpu.sync_copy(msg.at[pl.ds(off, GCS)], rows_v)
        pltpu.sync_copy(rows_v, acc.at[idx_v], add=True)
        return carry

    lax.fori_loop(0, epw // GCS, step, 0)
    plsc.subcore_barrier()

    @pl.when(c == 0)
    def _():
        pltpu.sync_copy(acc.at[pl.ds(s * ZR, ZR)], p0.at[pl.ds(s * ZR, ZR)])

    @pl.when(c == 1)
    def _():
        pltpu.sync_copy(acc.at[pl.ds(s * ZR, ZR)], p1.at[pl.ds(s * ZR, ZR)])


def _sc_scatter(msg, dst_idx, zblk):
    mesh = plsc.VectorSubcoreMesh(core_axis_name="c", subcore_axis_name="s")
    return pl.kernel(
        _scatter_body,
        out_type=(
            jax.ShapeDtypeStruct((NP, D), jnp.float32),
            jax.ShapeDtypeStruct((NP, D), jnp.float32),
        ),
        mesh=mesh,
        scratch_types=[
            pltpu.VMEM_SHARED((NP, D), jnp.float32),
            pltpu.VMEM((GCS,), jnp.int32),
            pltpu.VMEM((GCS, D), jnp.float32),
        ],
    )(msg, dst_idx, zblk)


def _init_kernel_body(types_ref, emb_ref, out_ref):
    t = types_ref[...]
    oh = (t == lax.broadcasted_iota(jnp.int32, (BN, A), 1)).astype(jnp.float32)
    out_ref[...] = jnp.dot(oh, emb_ref[...], preferred_element_type=jnp.float32)


def _tc_init(atom_types_2d, atom_emb):
    return pl.pallas_call(
        _init_kernel_body,
        grid=(N // BN,),
        in_specs=[
            pl.BlockSpec((BN, 1), lambda i: (i, 0)),
            pl.BlockSpec((A, D), lambda i: (0, 0)),
        ],
        out_specs=pl.BlockSpec((BN, D), lambda i: (i, 0)),
        out_shape=jax.ShapeDtypeStruct((N, D), jnp.float32),
    )(atom_types_2d, atom_emb)


def _edge_compute(src, tgt, bond, W1s, W1t, W1b, W2, Wa, hb, ab, b2,
                  msg_ref, bond_ref):
    hid = jnp.tanh(
        jnp.dot(src, W1s, preferred_element_type=jnp.float32)
        + jnp.dot(tgt, W1t, preferred_element_type=jnp.float32)
        + jnp.dot(bond, W1b, preferred_element_type=jnp.float32)
        + hb)
    nb = jnp.dot(hid, W2, preferred_element_type=jnp.float32) + b2
    u = jnp.tanh(jnp.dot(src, Wa, preferred_element_type=jnp.float32) + ab)
    msg_ref[...] = u * nb
    bond_ref[...] = bond + nb


def _edge_body_first(src_ref, tgt_ref, btyp_ref, bemb_ref, W1s_ref, W1t_ref,
                     W1b_ref, W2_ref, Wa_ref, hb_ref, ab_ref, b2_ref,
                     msg_ref, bond_ref):
    bt = btyp_ref[...]
    oh = (bt == lax.broadcasted_iota(jnp.int32, (BE, BC), 1)).astype(jnp.float32)
    bond = jnp.dot(oh, bemb_ref[...], preferred_element_type=jnp.float32)
    _edge_compute(src_ref[...], tgt_ref[...], bond, W1s_ref[...],
                  W1t_ref[...], W1b_ref[...], W2_ref[...], Wa_ref[...],
                  hb_ref[...], ab_ref[...], b2_ref[...], msg_ref, bond_ref)


def _edge_body_next(src_ref, tgt_ref, bin_ref, W1s_ref, W1t_ref,
                    W1b_ref, W2_ref, Wa_ref, hb_ref, ab_ref, b2_ref,
                    msg_ref, bond_ref):
    _edge_compute(src_ref[...], tgt_ref[...], bin_ref[...],
                  W1s_ref[...], W1t_ref[...], W1b_ref[...], W2_ref[...],
                  Wa_ref[...], hb_ref[...], ab_ref[...], b2_ref[...],
                  msg_ref, bond_ref)


def _full_spec(shape):
    return pl.BlockSpec(shape, lambda i: tuple(0 for _ in shape))


def _tc_edge(first, src_rows, dst_rows, bond_in, weights):
    body = _edge_body_first if first else _edge_body_next
    eb = pl.BlockSpec((BE, D), lambda i: (i, 0))
    if first:
        data_specs = [eb, eb, pl.BlockSpec((BE, 1), lambda i: (i, 0)),
                      _full_spec((BC, D))]
    else:
        data_specs = [eb, eb, eb]
    w_specs = [_full_spec((D, 2 * D)), _full_spec((D, 2 * D)),
               _full_spec((D, 2 * D)), _full_spec((2 * D, D)),
               _full_spec((D, D)), _full_spec((1, 2 * D)),
               _full_spec((1, D)), _full_spec((1, D))]
    return pl.pallas_call(
        body,
        grid=(E // BE,),
        in_specs=data_specs + w_specs,
        out_specs=(eb, eb),
        out_shape=(
            jax.ShapeDtypeStruct((E, D), jnp.float32),
            jax.ShapeDtypeStruct((E, D), jnp.float32),
        ),
    )(src_rows, dst_rows, *bond_in, *weights)


def _combine_body(a_ref, p0_ref, p1_ref, out_ref):
    out_ref[...] = a_ref[...] + p0_ref[...] + p1_ref[...]


def _tc_combine(atom, p0, p1):
    nb = pl.BlockSpec((BE, D), lambda i: (i, 0))
    return pl.pallas_call(
        _combine_body,
        grid=(N // BE,),
        in_specs=[nb, nb, nb],
        out_specs=nb,
        out_shape=jax.ShapeDtypeStruct((N, D), jnp.float32),
    )(atom, p0, p1)


def _final_body(a_ref, p0_ref, p1_ref, typ_ref, ng_ref, outW_ref, outb_ref,
                finW_ref, finb_ref, mtab_ref, out_ref):
    x = a_ref[...] + p0_ref[...] + p1_ref[...]
    h = jnp.maximum(
        jnp.dot(x, outW_ref[...], preferred_element_type=jnp.float32)
        + outb_ref[...], 0.0)
    e = jnp.dot(h, finW_ref[...], preferred_element_type=jnp.float32) + finb_ref[...]
    t = typ_ref[...]
    ohm = (t == lax.broadcasted_iota(jnp.int32, (BN, A), 1)).astype(jnp.float32)
    e = e + jnp.dot(ohm, mtab_ref[...], preferred_element_type=jnp.float32)
    g_row = ng_ref[...].reshape(1, BN)
    ohg = (jnp.broadcast_to(g_row, (G, BN))
           == lax.broadcasted_iota(jnp.int32, (G, BN), 0)).astype(jnp.float32)
    partial = jnp.dot(ohg, e, preferred_element_type=jnp.float32)

    @pl.when(pl.program_id(0) == 0)
    def _():
        out_ref[...] = jnp.zeros_like(out_ref)

    out_ref[...] += partial


def _tc_final(atom, p0, p1, atom_types_2d, ng_row, out_W, out_b2, final_W,
              final_b2, mtab):
    nb = pl.BlockSpec((BN, D), lambda i: (i, 0))
    return pl.pallas_call(
        _final_body,
        grid=(N // BN,),
        in_specs=[nb, nb, nb,
                  pl.BlockSpec((BN, 1), lambda i: (i, 0)),
                  pl.BlockSpec((1, 1, BN), lambda i: (i, 0, 0)),
                  _full_spec((D, H)), _full_spec((1, H)), _full_spec((H, 1)),
                  _full_spec((1, 1)), _full_spec((A, 1))],
        out_specs=pl.BlockSpec((G, 1), lambda i: (0, 0)),
        out_shape=jax.ShapeDtypeStruct((G, 1), jnp.float32),
    )(atom, p0, p1, atom_types_2d, ng_row, out_W, out_b2, final_W, final_b2,
      mtab)


def kernel(atom_emb, atom_mean_tab, bond_emb, msg_params, out_W, out_b,
           final_W, final_b, atom_types, bond_types, node_graph_indices,
           connectivity):
    inv = 1.0 / jnp.sqrt(1.0 + EPS)
    dst_idx = connectivity[:, 0].astype(jnp.int32)
    src_idx = connectivity[:, 1].astype(jnp.int32)
    atom_types_2d = atom_types.astype(jnp.int32).reshape(N, 1)
    bond_types_2d = bond_types.astype(jnp.int32).reshape(E, 1)
    ng_row = node_graph_indices.astype(jnp.int32).reshape(N // BN, 1, BN)
    zblk = jnp.zeros((ZR, D), jnp.float32)

    step_weights = []
    for p in msg_params:
        sa = p['atom_bn_gamma'] * inv
        ba = p['atom_bn_beta']
        sb = p['bond_bn_gamma'] * inv
        bb = p['bond_bn_beta']
        W1 = p['W1']
        W1s = sa[:, None] * W1[:D]
        W1t = sa[:, None] * W1[D:2 * D]
        W1b = sb[:, None] * W1[2 * D:]
        hb = (ba @ (W1[:D] + W1[D:2 * D]) + bb @ W1[2 * D:]).reshape(1, 2 * D)
        Wa = sa[:, None] * p['Wa']
        ab = (ba @ p['Wa']).reshape(1, D)
        b2 = p['b2'].reshape(1, D)
        step_weights.append((W1s, W1t, W1b, p['W2'], Wa, hb, ab, b2))

    atom = _tc_init(atom_types_2d, atom_emb)
    bond = None
    p0 = p1 = None
    for t in range(len(msg_params)):
        src_rows, dst_rows = _sc_gather(atom, src_idx, dst_idx)
        if t == 0:
            msg, bond = _tc_edge(True, src_rows, dst_rows,
                                 (bond_types_2d, bond_emb), step_weights[t])
        else:
            msg, bond = _tc_edge(False, src_rows, dst_rows, (bond,),
                                 step_weights[t])
        p0, p1 = _sc_scatter(msg, dst_idx, zblk)
        if t < len(msg_params) - 1:
            atom = _tc_combine(atom, p0, p1)

    return _tc_final(atom, p0, p1, atom_types_2d, ng_row, out_W,
                     out_b.reshape(1, H), final_W, final_b.reshape(1, 1),
                     atom_mean_tab)
